# Initial kernel scaffold; baseline (speedup 1.0000x reference)
import jax
import jax.numpy as jnp
from jax.experimental import pallas as pl


def kernel(X_seq, edge, Wxz, bxz, Whz, bhz, Wxr, bxr, Whr, bhr, Wxh, bxh, Whh, bhh, Whead, bhead):
    raise NotImplementedError("write your pallas kernel here")


# speedup vs baseline: 1.0165x; 1.0165x over previous
import jax
import jax.numpy as jnp
from jax.experimental import pallas as pl

_BLK = 2000


def _dot(a, b):
    return jax.lax.dot_general(a, b, (((1,), (0,)), ((), ())),
                               preferred_element_type=jnp.float32)


def _gates_kernel(x_ref, h_ref, sx_ref, sh_ref,
                  wx_ref, wsx_ref, wh_ref, wsh_ref, whh0_ref, b_ref,
                  z_ref, hr_ref, p_ref):
    acc = _dot(x_ref[...], wx_ref[...]) + _dot(sx_ref[...], wsx_ref[...])
    acc = acc + b_ref[...]
    ch = _dot(h_ref[...], wh_ref[...])
    csh = _dot(sh_ref[...], wsh_ref[...])
    pre_z = acc[:, :128] + ch[:, :128] + csh[:, :128]
    pre_r = acc[:, 128:256] + ch[:, 128:] + csh[:, 128:]
    z = jax.nn.sigmoid(pre_z)
    r = jax.nn.sigmoid(pre_r)
    hr = h_ref[...] * r
    p = acc[:, 256:] + _dot(hr, whh0_ref[...])
    z_ref[...] = z
    hr_ref[...] = hr
    p_ref[...] = p


def _update_kernel(p_ref, shr_ref, z_ref, h_ref,
                   whh1_ref, whead_ref, bhead_ref,
                   hnew_ref, u_ref):
    ht = jnp.tanh(p_ref[...] + _dot(shr_ref[...], whh1_ref[...]))
    z = z_ref[...]
    hnew = z * h_ref[...] + (1.0 - z) * ht
    hnew_ref[...] = hnew
    u_ref[...] = _dot(hnew, whead_ref[...]) + bhead_ref[...]


def kernel(X_seq, edge, Wxz, bxz, Whz, bhz, Wxr, bxr, Whr, bhr,
           Wxh, bxh, Whh, bhh, Whead, bhead):
    Tn, n, in_f = X_seq.shape
    Hdim = Whz.shape[2]
    out_f = Whead.shape[1]
    src, dst = edge[0], edge[1]

    deg = jax.ops.segment_sum(jnp.ones((edge.shape[1],), jnp.float32), src,
                              num_segments=n)
    dinv = jnp.where(deg > 0, jax.lax.rsqrt(jnp.where(deg > 0, deg, 1.0)), 0.0)
    wnorm = (-dinv[src] * dinv[dst])[:, None]

    inp = 16
    pad_cols = inp - in_f

    def padw(w):
        return jnp.pad(w, ((0, pad_cols), (0, 0)))

    WxC = jnp.concatenate([padw(Wxz[0]), padw(Wxr[0]), padw(Wxh[0])], axis=1)
    SxC = jnp.concatenate([padw(Wxz[1]), padw(Wxr[1]), padw(Wxh[1])], axis=1)
    WhC = jnp.concatenate([Whz[0], Whr[0]], axis=1)
    ShC = jnp.concatenate([Whz[1], Whr[1]], axis=1)
    bC = jnp.concatenate([bxz + bhz, bxr + bhr, bxh + bhh])[None, :]
    WheadP = jnp.pad(Whead, ((0, 0), (0, Hdim - out_f)))
    bheadP = jnp.pad(bhead, (0, Hdim - out_f))[None, :]

    nb = n // _BLK

    def spec_blk(c):
        return pl.BlockSpec((_BLK, c), lambda i: (i, 0))

    def spec_full(r, c):
        return pl.BlockSpec((r, c), lambda i: (0, 0))

    gates_call = pl.pallas_call(
        _gates_kernel,
        grid=(nb,),
        in_specs=[spec_blk(inp), spec_blk(Hdim), spec_blk(inp), spec_blk(Hdim),
                  spec_full(inp, 3 * Hdim), spec_full(inp, 3 * Hdim),
                  spec_full(Hdim, 2 * Hdim), spec_full(Hdim, 2 * Hdim),
                  spec_full(Hdim, Hdim), spec_full(1, 3 * Hdim)],
        out_specs=[spec_blk(Hdim)] * 3,
        out_shape=[jax.ShapeDtypeStruct((n, Hdim), jnp.float32)] * 3,
    )
    update_call = pl.pallas_call(
        _update_kernel,
        grid=(nb,),
        in_specs=[spec_blk(Hdim)] * 4 +
                 [spec_full(Hdim, Hdim), spec_full(Hdim, Hdim),
                  spec_full(1, Hdim)],
        out_specs=[spec_blk(Hdim)] * 2,
        out_shape=[jax.ShapeDtypeStruct((n, Hdim), jnp.float32)] * 2,
    )

    h = jnp.zeros((n, Hdim), jnp.float32)
    x_t = jnp.pad(X_seq[0], ((0, 0), (0, pad_cols)))
    outs = []
    for t in range(Tn):
        xh = jnp.concatenate([x_t, h], axis=1)
        sxh = jax.ops.segment_sum(wnorm * xh[src], dst, num_segments=n)
        sx = sxh[:, :inp]
        sh = sxh[:, inp:]
        z, hr, p = gates_call(x_t, h, sx, sh, WxC, SxC, WhC, ShC, Whh[0], bC)
        shr = jax.ops.segment_sum(wnorm * hr[src], dst, num_segments=n)
        h, u_pad = update_call(p, shr, z, h, Whh[1], WheadP, bheadP)
        u = u_pad[:, :out_f]
        outs.append(u)
        if t < Tn - 1:
            xn = X_seq[t + 1]
            dt = xn[:, 6] - X_seq[t][:, 6]
            v = (u - X_seq[t][:, 3:6]) / dt[:, None]
            xn = xn.at[:, 3:6].set(u).at[:, 8:11].set(v)
            x_t = jnp.pad(xn, ((0, 0), (0, pad_cols)))
    return jnp.stack(outs)


# speedup vs baseline: 1.0249x; 1.0083x over previous
import jax
import jax.numpy as jnp
from jax.experimental import pallas as pl

_BLK = 2000


def _dot(a, b):
    return jax.lax.dot_general(a, b, (((1,), (0,)), ((), ())),
                               preferred_element_type=jnp.float32)


def _gates_kernel(x_ref, h_ref, sx_ref, sh_ref,
                  wx_ref, wsx_ref, wh_ref, wsh_ref, whh0_ref, b_ref,
                  z_ref, hr_ref, p_ref):
    acc = _dot(x_ref[...], wx_ref[...]) + _dot(sx_ref[...], wsx_ref[...])
    acc = acc + b_ref[...]
    ch = _dot(h_ref[...], wh_ref[...])
    csh = _dot(sh_ref[...], wsh_ref[...])
    pre_z = acc[:, :128] + ch[:, :128] + csh[:, :128]
    pre_r = acc[:, 128:256] + ch[:, 128:] + csh[:, 128:]
    z = jax.nn.sigmoid(pre_z)
    r = jax.nn.sigmoid(pre_r)
    hr = h_ref[...] * r
    p = acc[:, 256:] + _dot(hr, whh0_ref[...])
    z_ref[...] = z
    hr_ref[...] = hr
    p_ref[...] = p


def _update_kernel(p_ref, shr_ref, z_ref, h_ref,
                   whh1_ref, whead_ref, bhead_ref,
                   hnew_ref, u_ref):
    ht = jnp.tanh(p_ref[...] + _dot(shr_ref[...], whh1_ref[...]))
    z = z_ref[...]
    hnew = z * h_ref[...] + (1.0 - z) * ht
    hnew_ref[...] = hnew
    u_ref[...] = _dot(hnew, whead_ref[...]) + bhead_ref[...]


def kernel(X_seq, edge, Wxz, bxz, Whz, bhz, Wxr, bxr, Whr, bhr,
           Wxh, bxh, Whh, bhh, Whead, bhead):
    Tn, n, in_f = X_seq.shape
    Hdim = Whz.shape[2]
    out_f = Whead.shape[1]
    src, dst = edge[0], edge[1]

    deg = jax.ops.segment_sum(jnp.ones((edge.shape[1],), jnp.float32), src,
                              num_segments=n)
    dinv = jnp.where(deg > 0, jax.lax.rsqrt(jnp.where(deg > 0, deg, 1.0)), 0.0)
    order = jnp.argsort(dst)
    src = src[order]
    dst = dst[order]
    wnorm = (-dinv[src] * dinv[dst])[:, None]

    inp = 16
    pad_cols = inp - in_f

    def padw(w):
        return jnp.pad(w, ((0, pad_cols), (0, 0)))

    WxC = jnp.concatenate([padw(Wxz[0]), padw(Wxr[0]), padw(Wxh[0])], axis=1)
    SxC = jnp.concatenate([padw(Wxz[1]), padw(Wxr[1]), padw(Wxh[1])], axis=1)
    WhC = jnp.concatenate([Whz[0], Whr[0]], axis=1)
    ShC = jnp.concatenate([Whz[1], Whr[1]], axis=1)
    bC = jnp.concatenate([bxz + bhz, bxr + bhr, bxh + bhh])[None, :]
    WheadP = jnp.pad(Whead, ((0, 0), (0, Hdim - out_f)))
    bheadP = jnp.pad(bhead, (0, Hdim - out_f))[None, :]

    nb = n // _BLK

    def spec_blk(c):
        return pl.BlockSpec((_BLK, c), lambda i: (i, 0))

    def spec_full(r, c):
        return pl.BlockSpec((r, c), lambda i: (0, 0))

    gates_call = pl.pallas_call(
        _gates_kernel,
        grid=(nb,),
        in_specs=[spec_blk(inp), spec_blk(Hdim), spec_blk(inp), spec_blk(Hdim),
                  spec_full(inp, 3 * Hdim), spec_full(inp, 3 * Hdim),
                  spec_full(Hdim, 2 * Hdim), spec_full(Hdim, 2 * Hdim),
                  spec_full(Hdim, Hdim), spec_full(1, 3 * Hdim)],
        out_specs=[spec_blk(Hdim)] * 3,
        out_shape=[jax.ShapeDtypeStruct((n, Hdim), jnp.float32)] * 3,
    )
    update_call = pl.pallas_call(
        _update_kernel,
        grid=(nb,),
        in_specs=[spec_blk(Hdim)] * 4 +
                 [spec_full(Hdim, Hdim), spec_full(Hdim, Hdim),
                  spec_full(1, Hdim)],
        out_specs=[spec_blk(Hdim)] * 2,
        out_shape=[jax.ShapeDtypeStruct((n, Hdim), jnp.float32)] * 2,
    )

    h = jnp.zeros((n, Hdim), jnp.float32)
    x_t = jnp.pad(X_seq[0], ((0, 0), (0, pad_cols)))
    outs = []
    for t in range(Tn):
        xh = jnp.concatenate([x_t, h], axis=1)
        sxh = jax.ops.segment_sum(wnorm * xh[src], dst, num_segments=n,
                                  indices_are_sorted=True)
        sx = sxh[:, :inp]
        sh = sxh[:, inp:]
        z, hr, p = gates_call(x_t, h, sx, sh, WxC, SxC, WhC, ShC, Whh[0], bC)
        shr = jax.ops.segment_sum(wnorm * hr[src], dst, num_segments=n,
                                  indices_are_sorted=True)
        h, u_pad = update_call(p, shr, z, h, Whh[1], WheadP, bheadP)
        u = u_pad[:, :out_f]
        outs.append(u)
        if t < Tn - 1:
            xn = X_seq[t + 1]
            dt = xn[:, 6] - X_seq[t][:, 6]
            v = (u - X_seq[t][:, 3:6]) / dt[:, None]
            xn = xn.at[:, 3:6].set(u).at[:, 8:11].set(v)
            x_t = jnp.pad(xn, ((0, 0), (0, pad_cols)))
    return jnp.stack(outs)
